# baseline (speedup 1.0000x reference)
import functools

import jax
import jax.numpy as jnp
from jax import lax
from jax.experimental import pallas as pl
from jax.experimental.pallas import tpu as pltpu
from jax.experimental.pallas import tpu_sc as plsc

HIDDEN = 32

NUM_CORES = 2
NUM_SUBCORES = 16
NW = NUM_CORES * NUM_SUBCORES

BB = 128
L_BLK = 8


def _make_kernel(batch: int, hist: int, vocab: int):
    hist_pad = ((hist + L_BLK - 1) // L_BLK) * L_BLK
    b_per_w = batch // NW
    n_bblk = b_per_w // BB
    n_lblk = hist_pad // L_BLK
    n_unit = n_lblk * n_bblk
    packed = vocab * HIDDEN // 128
    tail_l = hist - (n_lblk - 1) * L_BLK
    first_tail_unit = (n_lblk - 1) * n_bblk

    mesh = plsc.VectorSubcoreMesh(core_axis_name="c", subcore_axis_name="s")

    @functools.partial(
        pl.kernel,
        mesh=mesh,
        out_type=jax.ShapeDtypeStruct((hist, HIDDEN, batch), jnp.float32),
        scratch_types=[
            pltpu.VMEM((L_BLK, BB), jnp.int32),
            pltpu.VMEM((L_BLK, BB), jnp.int32),
            pltpu.VMEM((L_BLK, BB), jnp.int32),
            pltpu.VMEM((2, BB, 128), jnp.float32),
            pltpu.VMEM((L_BLK, HIDDEN, BB), jnp.float32),
            pltpu.SemaphoreType.DMA,
            pltpu.SemaphoreType.DMA,
        ],
        compiler_params=pltpu.CompilerParams(needs_layout_passes=False),
    )
    def gather_kernel(idx_hbm, table_hbm, out_hbm, idx_v, pidx_v, sub_v,
                      gbuf, obuf, sem_g, sem_o):
        wid = lax.axis_index("s") * NUM_CORES + lax.axis_index("c")
        b0w = wid * b_per_w
        iota = lax.iota(jnp.int32, 16)

        def fire_gather(l):
            pltpu.async_copy(
                table_hbm.at[pidx_v.at[l]], gbuf.at[lax.rem(l, 2)], sem_g)

        def wait_gather():
            pltpu.make_async_copy(
                table_hbm.at[pidx_v.at[0]], gbuf.at[0], sem_g).wait()

        def wait_out(j, carry):
            pltpu.make_async_copy(
                obuf.at[0], out_hbm.at[0, :, pl.ds(0, BB)], sem_o).wait()
            return carry

        def unit(u, carry):
            lb = u // n_bblk
            bb = lax.rem(u, n_bblk)
            l0 = pl.multiple_of(lb * L_BLK, L_BLK)
            b_abs = pl.multiple_of(b0w + bb * BB, BB)
            lmax = jnp.minimum(L_BLK, hist - l0)

            prev_writes = jnp.where(u > first_tail_unit, tail_l,
                                    jnp.where(u > 0, L_BLK, 0))
            lax.fori_loop(0, prev_writes, wait_out, 0)

            pltpu.sync_copy(
                idx_hbm.at[pl.ds(l0, L_BLK), pl.ds(b_abs, BB)], idx_v)

            def prep(r, c):
                for cg in range(BB // 16):
                    v = idx_v[r, pl.ds(cg * 16, 16)]
                    pidx_v[r, pl.ds(cg * 16, 16)] = \
                        lax.shift_right_logical(v, 2)
                    sub_v[r, pl.ds(cg * 16, 16)] = lax.bitwise_and(v, 3)
                return c

            lax.fori_loop(0, L_BLK, prep, 0)

            fire_gather(0)

            def lbody(l, c):
                wait_gather()

                @pl.when(l + 1 < lmax)
                def _fire_next():
                    fire_gather(l + 1)

                g2 = gbuf.at[lax.rem(l, 2)]
                for bb16 in range(BB // 16):
                    sv = sub_v[l, pl.ds(bb16 * 16, 16)]
                    rows = iota + (bb16 * 16)
                    base = sv * HIDDEN
                    for d in range(HIDDEN):
                        val = plsc.load_gather(g2, [rows, base + d])
                        obuf[l, d, pl.ds(bb16 * 16, 16)] = val

                pltpu.async_copy(
                    obuf.at[l],
                    out_hbm.at[l0 + l, :, pl.ds(b_abs, BB)], sem_o)
                return c

            lax.fori_loop(0, lmax, lbody, 0)
            return carry

        lax.fori_loop(0, n_unit, unit, 0)

        lax.fori_loop(0, tail_l, wait_out, 0)

    return gather_kernel


def kernel(indices, table):
    batch, hist = indices.shape
    vocab = table.shape[0]
    hist_pad = ((hist + L_BLK - 1) // L_BLK) * L_BLK
    idx_t = jnp.pad(indices, ((0, 0), (0, hist_pad - hist))).T
    table_p = table.reshape(vocab * HIDDEN // 128, 128)
    res = _make_kernel(batch, hist, vocab)(idx_t, table_p)
    return jnp.transpose(res, (2, 0, 1))
